# baseline (speedup 1.0000x reference)
import functools

import jax
import jax.numpy as jnp
from jax import lax
from jax.experimental import pallas as pl
from jax.experimental.pallas import tpu as pltpu
from jax.experimental.pallas import tpu_sc as plsc

_N_ITEM = 100000
_H = 64
_B = 1024
_L = 50
_NC, _NS = 2, 16
_NW = _NC * _NS
_BPW = _B // _NW
_INV = 1.0 / (3 * _L)
_LP = 56
_SEG = 3 * _LP

_TC = 2048
_NT = (_N_ITEM + _TC - 1) // _TC


def _pool_body(item_i, entity_i, word_i, item_t, entity_t, word_t, out_hbm,
               idx0, idx1, idx2, buf0, buf1, acc, sem0, sem1):
    wid = lax.axis_index("s") * _NC + lax.axis_index("c")
    base = wid * _BPW
    pltpu.sync_copy(item_i.at[pl.ds(base * _LP, _BPW * _LP)], idx0)
    pltpu.sync_copy(entity_i.at[pl.ds(base * _LP, _BPW * _LP)], idx1)
    pltpu.sync_copy(word_i.at[pl.ds(base * _LP, _BPW * _LP)], idx2)

    tables = (item_t, entity_t, word_t)
    idxs = (idx0, idx1, idx2)
    bufs = (buf0, buf1)
    sems = (sem0, sem1)
    inflight = [None, None]

    def _gather_row(b):
        pend = []
        for t in range(3):
            pend.append(pltpu.async_copy(
                tables[t].at[idxs[t].at[pl.ds(b * _LP, _LP)]],
                bufs[b % 2].at[pl.ds(t * _LP, _LP)],
                sems[b % 2]))
        inflight[b % 2] = pend

    def _reduce_row(b):
        for d in inflight[b % 2]:
            d.wait()
        buf = bufs[b % 2]
        tot = (jnp.zeros((16,), jnp.float32),) * 4
        for t in range(3):
            def body(r, carry, _t=t):
                return tuple(carry[c] + buf[_t * _LP + r, pl.ds(16 * c, 16)]
                             for c in range(4))
            tot = lax.fori_loop(0, _L, body, tot)
        for c in range(4):
            acc[pl.ds(b * _H + 16 * c, 16)] = tot[c]

    for s in range(_BPW + 1):
        if s < _BPW:
            _gather_row(s)
        if s > 0:
            _reduce_row(s - 1)

    pltpu.sync_copy(acc, out_hbm.at[pl.ds(base * _H, _BPW * _H)])


@functools.lru_cache(maxsize=1)
def _make_pool():
  return pl.kernel(
    _pool_body,
    out_type=jax.ShapeDtypeStruct((_B * _H,), jnp.float32),
    mesh=plsc.VectorSubcoreMesh(core_axis_name="c", subcore_axis_name="s",
                                num_cores=_NC, num_subcores=_NS),
    scratch_types=[
        pltpu.VMEM((_BPW * _LP,), jnp.int32),
        pltpu.VMEM((_BPW * _LP,), jnp.int32),
        pltpu.VMEM((_BPW * _LP,), jnp.int32),
        pltpu.VMEM((_SEG, _H), jnp.float32),
        pltpu.VMEM((_SEG, _H), jnp.float32),
        pltpu.VMEM((_BPW * _H,), jnp.float32),
        pltpu.SemaphoreType.DMA,
        pltpu.SemaphoreType.DMA,
    ],
    compiler_params=pltpu.CompilerParams(use_tc_tiling_on_sc=False),
  )


def _stats_body(u_ref, wt_ref, b_ref, m_out, z_out, m_acc, z_acc):
    j = pl.program_id(0)

    @pl.when(j == 0)
    def _():
        m_acc[...] = jnp.full_like(m_acc, -jnp.inf)
        z_acc[...] = jnp.zeros_like(z_acc)

    u = u_ref[...] * _INV
    x = jnp.dot(u, wt_ref[...], preferred_element_type=jnp.float32) + b_ref[...]
    col = j * _TC + lax.broadcasted_iota(jnp.int32, x.shape, 1)
    x = jnp.where(col < _N_ITEM, x, -jnp.inf)
    m_tile = jnp.max(x, axis=1, keepdims=True)
    m_new = jnp.maximum(m_acc[...], m_tile)
    z_acc[...] = (z_acc[...] * jnp.exp(m_acc[...] - m_new)
                  + jnp.sum(jnp.exp(x - m_new), axis=1, keepdims=True))
    m_acc[...] = m_new

    @pl.when(j == _NT - 1)
    def _():
        m_out[...] = m_acc[...]
        z_out[...] = z_acc[...]


_stats_call = pl.pallas_call(
    _stats_body,
    grid=(_NT,),
    in_specs=[
        pl.BlockSpec((_B, _H), lambda j: (0, 0)),
        pl.BlockSpec((_H, _TC), lambda j: (0, j)),
        pl.BlockSpec((1, _TC), lambda j: (0, j)),
    ],
    out_specs=[
        pl.BlockSpec((_B, 1), lambda j: (0, 0)),
        pl.BlockSpec((_B, 1), lambda j: (0, 0)),
    ],
    out_shape=[
        jax.ShapeDtypeStruct((_B, 1), jnp.float32),
        jax.ShapeDtypeStruct((_B, 1), jnp.float32),
    ],
    scratch_shapes=[
        pltpu.VMEM((_B, 1), jnp.float32),
        pltpu.VMEM((_B, 1), jnp.float32),
    ],
    compiler_params=pltpu.CompilerParams(dimension_semantics=("arbitrary",)),
)


def _softmax_body(u_ref, wt_ref, b_ref, m_ref, z_ref, lab_ref,
                  out_ref, loss_ref, s_acc, p_acc):
    j = pl.program_id(0)

    @pl.when(j == 0)
    def _():
        s_acc[...] = jnp.zeros_like(s_acc)
        p_acc[...] = jnp.zeros_like(p_acc)

    u = u_ref[...] * _INV
    x = jnp.dot(u, wt_ref[...], preferred_element_type=jnp.float32) + b_ref[...]
    col = j * _TC + lax.broadcasted_iota(jnp.int32, x.shape, 1)
    ok = col < _N_ITEM
    p = jnp.where(ok, jnp.exp(x - m_ref[...]) / z_ref[...], 0.0)
    out_ref[...] = p
    s_acc[...] += jnp.sum(jnp.where(ok, jnp.exp(p), 0.0), axis=1, keepdims=True)
    p_acc[...] += jnp.sum(jnp.where(col == lab_ref[...], p, 0.0),
                          axis=1, keepdims=True)

    @pl.when(j == _NT - 1)
    def _():
        loss_ref[0, 0] = jnp.mean(jnp.log(s_acc[...]) - p_acc[...])


_softmax_call = pl.pallas_call(
    _softmax_body,
    grid=(_NT,),
    in_specs=[
        pl.BlockSpec((_B, _H), lambda j: (0, 0)),
        pl.BlockSpec((_H, _TC), lambda j: (0, j)),
        pl.BlockSpec((1, _TC), lambda j: (0, j)),
        pl.BlockSpec((_B, 1), lambda j: (0, 0)),
        pl.BlockSpec((_B, 1), lambda j: (0, 0)),
        pl.BlockSpec((_B, 1), lambda j: (0, 0)),
    ],
    out_specs=[
        pl.BlockSpec((_B, _TC), lambda j: (0, j)),
        pl.BlockSpec(memory_space=pltpu.SMEM, block_shape=(1, 1),
                     index_map=lambda j: (0, 0)),
    ],
    out_shape=[
        jax.ShapeDtypeStruct((_B, _N_ITEM), jnp.float32),
        jax.ShapeDtypeStruct((1, 1), jnp.float32),
    ],
    scratch_shapes=[
        pltpu.VMEM((_B, 1), jnp.float32),
        pltpu.VMEM((_B, 1), jnp.float32),
    ],
    compiler_params=pltpu.CompilerParams(dimension_semantics=("arbitrary",)),
)


def kernel(item_idx, entity_idx, word_idx, labels, item_table, entity_table,
           word_table, W_rec, b_rec):
    def _prep(idx):
        idx = jnp.pad(idx.astype(jnp.int32), ((0, 0), (0, _LP - _L)))
        return jnp.reshape(idx, (_B * _LP,))

    item_f = _prep(item_idx)
    entity_f = _prep(entity_idx)
    word_f = _prep(word_idx)

    u = jnp.reshape(
        _make_pool()(item_f, entity_f, word_f,
                     item_table, entity_table, word_table),
        (_B, _H))

    wt = W_rec.T
    b2 = jnp.reshape(b_rec, (1, _N_ITEM))
    lab2 = jnp.reshape(labels.astype(jnp.int32), (_B, 1))

    m, z = _stats_call(u, wt, b2)
    logits, loss = _softmax_call(u, wt, b2, m, z, lab2)
    return logits, labels, loss[0, 0]
